# baseline (speedup 1.0000x reference)
import functools

import jax
import jax.numpy as jnp
from jax import lax
from jax.experimental import pallas as pl
from jax.experimental.pallas import tpu as pltpu
from jax.experimental.pallas import tpu_sc as plsc

_N = 10000
_E = 320000
_G = 64
_NP = 10240
_SD = 256
_ED = 32

_NB = 2048
_EB = 2048
_EP = 329728
_EPW = _E // 32
_EPT = _EP // 16
_CH = 56
_NCH = _EPT // _CH
_ROWS_PT = _NP // 16


def _mesh():
    return plsc.VectorSubcoreMesh(core_axis_name="c", subcore_axis_name="s")


def _k1_body(pt_blk, b_blk, cnt_ref, ps_ref):
    i = pl.program_id(0)
    oh = (b_blk[...] == lax.broadcasted_iota(jnp.int32, (_NB, _G), 1)
          ).astype(jnp.float32)
    cnt_p = jnp.sum(oh, axis=0, keepdims=True)
    ps_p = lax.dot_general(pt_blk[...], oh, (((1,), (0,)), ((), ())),
                           preferred_element_type=jnp.float32)

    @pl.when(i == 0)
    def _():
        cnt_ref[...] = cnt_p
        ps_ref[...] = ps_p

    @pl.when(i != 0)
    def _():
        cnt_ref[...] = cnt_ref[...] + cnt_p
        ps_ref[...] = ps_ref[...] + ps_p


def _k1_stats(pos_t, batch2):
    return pl.pallas_call(
        _k1_body,
        grid=(_NP // _NB,),
        in_specs=[pl.BlockSpec((3, _NB), lambda i: (0, i)),
                  pl.BlockSpec((_NB, 1), lambda i: (i, 0))],
        out_specs=[pl.BlockSpec((1, _G), lambda i: (0, 0)),
                   pl.BlockSpec((3, _G), lambda i: (0, 0))],
        out_shape=[jax.ShapeDtypeStruct((1, _G), jnp.float32),
                   jax.ShapeDtypeStruct((3, _G), jnp.float32)],
    )(pos_t, batch2)


def _k2_body(x_blk, b_blk, pt_blk, t_ref, cnt_ref, ps_ref, Wta, bta, Wam,
             bam, Wat, bat, W1, W2, s0_ref, pct_ref, p12_ref):
    oh = (b_blk[...] == lax.broadcasted_iota(jnp.int32, (_NB, _G), 1)
          ).astype(jnp.float32)
    meanp = ps_ref[...] / jnp.maximum(cnt_ref[...], 1.0)
    pct_ref[...] = pt_blk[...] - lax.dot_general(
        meanp, oh, (((1,), (1,)), ((), ())),
        preferred_element_type=jnp.float32)
    ta = jnp.dot(t_ref[...], Wta[...],
                 preferred_element_type=jnp.float32) + bta[...]
    tn = jnp.dot(oh, ta, preferred_element_type=jnp.float32)
    s_pre = (jnp.dot(x_blk[...], Wam[...],
                     preferred_element_type=jnp.float32) + bam[...] + tn)
    s0 = jnp.dot(s_pre, Wat[...],
                 preferred_element_type=jnp.float32) + bat[...]
    s0_ref[...] = s0
    p1 = jnp.dot(s0, W1[...], preferred_element_type=jnp.float32)
    p2 = jnp.dot(s0, W2[...], preferred_element_type=jnp.float32)
    p12_ref[0] = p1[:, :128]
    p12_ref[1] = p1[:, 128:]
    p12_ref[2] = p2[:, :128]
    p12_ref[3] = p2[:, 128:]


def _k2_node(x_p, batch2, pos_t, t, cnt, ps_t, Wta, bta, Wam, bam, Wat, bat,
             W1, W2):
    full = lambda shape: pl.BlockSpec(shape, lambda i: tuple(0 for _ in shape))
    return pl.pallas_call(
        _k2_body,
        grid=(_NP // _NB,),
        in_specs=[pl.BlockSpec((_NB, 128), lambda i: (i, 0)),
                  pl.BlockSpec((_NB, 1), lambda i: (i, 0)),
                  pl.BlockSpec((3, _NB), lambda i: (0, i)),
                  full((_G, 1)), full((1, _G)), full((3, _G)),
                  full((1, _SD)), full((1, _SD)),
                  full((128, _SD)), full((1, _SD)),
                  full((_SD, _SD)), full((1, _SD)),
                  full((_SD, _SD)), full((_SD, _SD))],
        out_specs=[pl.BlockSpec((_NB, _SD), lambda i: (i, 0)),
                   pl.BlockSpec((3, _NB), lambda i: (0, i)),
                   pl.BlockSpec((4, _NB, 128), lambda i: (0, i, 0))],
        out_shape=[jax.ShapeDtypeStruct((_NP, _SD), jnp.float32),
                   jax.ShapeDtypeStruct((3, _NP), jnp.float32),
                   jax.ShapeDtypeStruct((4, _NP, 128), jnp.float32)],
    )(x_p, batch2, pos_t, t, cnt, ps_t, Wta, bta, Wam, bam, Wat, bat, W1, W2)


def _geom_body(posx, posy, posz, src_h, dst_h, d2_h, a_h, degp_h, px, py, pz,
               srcv, dstv, d2v, av, degv):
    c = lax.axis_index("c")
    s = lax.axis_index("s")
    w = s * 2 + c
    pltpu.sync_copy(posx, px)
    pltpu.sync_copy(posy, py)
    pltpu.sync_copy(posz, pz)
    base = w * _EPW
    pltpu.sync_copy(src_h.at[pl.ds(base, _EPW)], srcv)
    pltpu.sync_copy(dst_h.at[pl.ds(base, _EPW)], dstv)
    zeros16 = jnp.zeros((16,), jnp.float32)

    def zbody(i, carry):
        degv[pl.ds(i * 16, 16)] = zeros16
        return carry

    lax.fori_loop(0, _NP // 16, zbody, 0)
    ones16 = jnp.ones((16,), jnp.float32)

    def body(j, carry):
        si = srcv[pl.ds(j * 16, 16)]
        di = dstv[pl.ds(j * 16, 16)]
        xs = plsc.load_gather(px, [si])
        ys = plsc.load_gather(py, [si])
        zs = plsc.load_gather(pz, [si])
        xd = plsc.load_gather(px, [di])
        yd = plsc.load_gather(py, [di])
        zd = plsc.load_gather(pz, [di])
        dx = xd - xs
        dy = yd - ys
        dz = zd - zs
        d2v[pl.ds(j * 16, 16)] = dx * dx + dy * dy + dz * dz
        av[pl.ds(j * 16, 16)] = xs * xd + ys * yd + zs * zd
        plsc.addupdate_scatter(degv, [di], ones16)
        return carry

    lax.fori_loop(0, _EPW // 16, body, 0)
    pltpu.sync_copy(d2v, d2_h.at[pl.ds(base, _EPW)])
    pltpu.sync_copy(av, a_h.at[pl.ds(base, _EPW)])
    pltpu.sync_copy(degv, degp_h.at[pl.ds(w * _NP, _NP)])


def _k3_geom(posx, posy, posz, src, dst):
    fn = functools.partial(
        pl.kernel,
        out_type=(jax.ShapeDtypeStruct((_E,), jnp.float32),
                  jax.ShapeDtypeStruct((_E,), jnp.float32),
                  jax.ShapeDtypeStruct((32 * _NP,), jnp.float32)),
        mesh=_mesh(),
        compiler_params=pltpu.CompilerParams(needs_layout_passes=False),
        scratch_types=[pltpu.VMEM((_NP,), jnp.float32),
                       pltpu.VMEM((_NP,), jnp.float32),
                       pltpu.VMEM((_NP,), jnp.float32),
                       pltpu.VMEM((_EPW,), jnp.int32),
                       pltpu.VMEM((_EPW,), jnp.int32),
                       pltpu.VMEM((_EPW,), jnp.float32),
                       pltpu.VMEM((_EPW,), jnp.float32),
                       pltpu.VMEM((_NP,), jnp.float32)],
    )(_geom_body)
    return fn(posx, posy, posz, src, dst)


def _k4_body(ea_blk, d2_blk, a_blk, be_blk, t_ref, Wtb, btb, Wbm, bbm, Wbt,
             bbt, WmE, wd, wa, bm, c_ref):
    oh = (be_blk[...] == lax.broadcasted_iota(jnp.int32, (_EB, _G), 1)
          ).astype(jnp.float32)
    tb = jnp.dot(t_ref[...], Wtb[...],
                 preferred_element_type=jnp.float32) + btb[...]
    te = jnp.dot(oh, tb, preferred_element_type=jnp.float32)
    e1 = (jnp.dot(ea_blk[...], Wbm[...],
                  preferred_element_type=jnp.float32) + bbm[...] + te)
    e2 = jnp.dot(e1, Wbt[...], preferred_element_type=jnp.float32) + bbt[...]
    dd = jnp.sqrt(jnp.maximum(d2_blk[...], 1e-6))
    cc = (jnp.dot(e2, WmE[...], preferred_element_type=jnp.float32)
          + dd * wd[...] + a_blk[...] * wa[...] + bm[...])
    rid = (pl.program_id(1) * _EB
           + lax.broadcasted_iota(jnp.int32, (_EB, 1), 0))
    c_ref[0] = jnp.where(rid < _E, cc, -1e6)


def _k4_cterm(ea, d2c, ac, be2, t, Wtb, btb, Wbm, bbm, Wbt, bbt, WmE, wd, wa,
              bm):
    full = lambda shape: pl.BlockSpec(shape, lambda c, i: tuple(0 for _ in shape))
    return pl.pallas_call(
        _k4_body,
        grid=(2, _EP // _EB),
        in_specs=[pl.BlockSpec((_EB, 8), lambda c, i: (i, 0)),
                  pl.BlockSpec((_EB, 1), lambda c, i: (i, 0)),
                  pl.BlockSpec((_EB, 1), lambda c, i: (i, 0)),
                  pl.BlockSpec((_EB, 1), lambda c, i: (i, 0)),
                  full((_G, 1)), full((1, _ED)), full((1, _ED)),
                  full((8, _ED)), full((1, _ED)),
                  full((_ED, _ED)), full((1, _ED)),
                  pl.BlockSpec((_ED, 128), lambda c, i: (0, c)),
                  pl.BlockSpec((1, 128), lambda c, i: (0, c)),
                  pl.BlockSpec((1, 128), lambda c, i: (0, c)),
                  pl.BlockSpec((1, 128), lambda c, i: (0, c))],
        out_specs=pl.BlockSpec((1, _EB, 128), lambda c, i: (c, i, 0)),
        out_shape=jax.ShapeDtypeStruct((2, _EP, 128), jnp.float32),
    )(ea, d2c, ac, be2, t, Wtb, btb, Wbm, bbm, Wbt, bbt, WmE, wd, wa, bm)


def _edge_body(p12f, clf, gidx, dstp, out, si0, si1, di0, di1, dp0, dp1,
               b1a, b1b, b2a, b2b, bca, bcb, agg, sx0, sx1, sg0, sg1):
    c = lax.axis_index("c")
    s = lax.axis_index("s")
    sis = (si0, si1)
    dis = (di0, di1)
    dps = (dp0, dp1)
    b1s = (b1a, b1b)
    b2s = (b2a, b2b)
    bcs = (bca, bcb)
    sxs = (sx0, sx1)
    sgs = (sg0, sg1)
    zrow = jnp.zeros((16,), jnp.float32)

    def zb_body(r, carry):
        for q in range(8):
            b1a[r, pl.ds(q * 16, 16)] = zrow
        return carry

    lax.fori_loop(0, _CH, zb_body, 0)
    rbase = s * _ROWS_PT
    for i in range(_ROWS_PT // _CH):
        pltpu.sync_copy(b1a, agg.at[pl.ds(rbase + i * _CH, _CH)])
    _rem = _ROWS_PT - (_ROWS_PT // _CH) * _CH
    pltpu.sync_copy(b1a.at[pl.ds(0, _rem)],
                    agg.at[pl.ds(rbase + (_ROWS_PT // _CH) * _CH, _rem)])
    plsc.subcore_barrier()
    ebase = s * _EPT
    coff = c * _EP
    ioff = (c * 16 + s) * _NCH * 2 * _CH

    def idx_start(k, sb):
        pltpu.make_async_copy(gidx.at[pl.ds(ioff + k * 2 * _CH, _CH)],
                              sis[sb], sxs[sb]).start()
        pltpu.make_async_copy(gidx.at[pl.ds(ioff + k * 2 * _CH + _CH, _CH)],
                              dis[sb], sxs[sb]).start()
        pltpu.make_async_copy(dstp.at[pl.ds(ebase + k * _CH, _CH)],
                              dps[sb], sxs[sb]).start()

    def idx_wait(sb):
        pltpu.make_async_copy(dstp.at[pl.ds(0, _CH)], sis[sb],
                              sxs[sb]).wait()
        pltpu.make_async_copy(dstp.at[pl.ds(0, _CH)], dis[sb],
                              sxs[sb]).wait()
        pltpu.make_async_copy(dstp.at[pl.ds(0, _CH)], dps[sb],
                              sxs[sb]).wait()

    def gather_start(k, sb):
        pltpu.make_async_copy(p12f.at[sis[sb]], b1s[sb], sgs[sb]).start()
        pltpu.make_async_copy(p12f.at[dis[sb]], b2s[sb], sgs[sb]).start()
        pltpu.make_async_copy(clf.at[pl.ds(coff + ebase + k * _CH, _CH)],
                              bcs[sb], sgs[sb]).start()

    def gather_wait(sb):
        pltpu.make_async_copy(clf.at[pl.ds(0, _CH)], bcs[sb],
                              sgs[sb]).wait()
        pltpu.make_async_copy(clf.at[pl.ds(0, _CH)], b1s[sb],
                              sgs[sb]).wait()
        pltpu.make_async_copy(clf.at[pl.ds(0, _CH)], b2s[sb],
                              sgs[sb]).wait()

    idx_start(0, 0)
    idx_wait(0)
    gather_start(0, 0)
    idx_start(1, 1)

    def outer(k0, carry):
        for b in range(2):
            k = k0 * 2 + b
            nk = k + 1

            @pl.when(nk < _NCH)
            def _():
                idx_wait(1 - b)
                gather_start(nk, 1 - b)

            gather_wait(b)

            def crow(r, carry2):
                for q in range(8):
                    x = (b1s[b][r, pl.ds(q * 16, 16)]
                         + b2s[b][r, pl.ds(q * 16, 16)]
                         + bcs[b][r, pl.ds(q * 16, 16)])
                    x = jnp.maximum(x, -60.0)
                    b1s[b][r, pl.ds(q * 16, 16)] = x / (1.0 + jnp.exp(-x))
                return carry2

            lax.fori_loop(0, _CH, crow, 0)
            pltpu.sync_copy(b1s[b], agg.at[dps[b]], add=True)

            @pl.when(k + 2 < _NCH)
            def _():
                idx_start(k + 2, b)
        return carry

    lax.fori_loop(0, _NCH // 2, outer, 0)
    plsc.subcore_barrier()
    pltpu.sync_copy(agg.at[pl.ds(rbase, _ROWS_PT)],
                    out.at[c, pl.ds(rbase, _ROWS_PT)])


def _k5_edge(p12f, clf, gidx, dstp):
    fn = functools.partial(
        pl.kernel,
        out_type=jax.ShapeDtypeStruct((2, _NP, 128), jnp.float32),
        mesh=_mesh(),
        compiler_params=pltpu.CompilerParams(needs_layout_passes=False),
        scratch_types=[pltpu.VMEM((_CH,), jnp.int32),
                       pltpu.VMEM((_CH,), jnp.int32),
                       pltpu.VMEM((_CH,), jnp.int32),
                       pltpu.VMEM((_CH,), jnp.int32),
                       pltpu.VMEM((_CH,), jnp.int32),
                       pltpu.VMEM((_CH,), jnp.int32),
                       pltpu.VMEM((_CH, 128), jnp.float32),
                       pltpu.VMEM((_CH, 128), jnp.float32),
                       pltpu.VMEM((_CH, 128), jnp.float32),
                       pltpu.VMEM((_CH, 128), jnp.float32),
                       pltpu.VMEM((_CH, 128), jnp.float32),
                       pltpu.VMEM((_CH, 128), jnp.float32),
                       pltpu.VMEM_SHARED((_NP, 128), jnp.float32),
                       pltpu.SemaphoreType.DMA,
                       pltpu.SemaphoreType.DMA,
                       pltpu.SemaphoreType.DMA,
                       pltpu.SemaphoreType.DMA],
    )(_edge_body)
    return fn(p12f, clf, gidx, dstp)


def _k6_body(s_blk, ag_blk, dg_blk, ones32, Wu, bu, W1, W2, sn_ref, p12_ref):
    deg = lax.dot_general(dg_blk[...], ones32[...], (((0,), (0,)), ((), ())),
                          preferred_element_type=jnp.float32)
    deginv = 1.0 / jnp.maximum(deg, 1.0)
    agg = jnp.concatenate([ag_blk[0], ag_blk[1]], axis=1) * deginv
    cat = jnp.concatenate([s_blk[...], agg], axis=1)
    upd = jax.nn.silu(jnp.dot(cat, Wu[...],
                              preferred_element_type=jnp.float32) + bu[...])
    sn = s_blk[...] + upd
    sn_ref[...] = sn
    p1 = jnp.dot(sn, W1[...], preferred_element_type=jnp.float32)
    p2 = jnp.dot(sn, W2[...], preferred_element_type=jnp.float32)
    p12_ref[0] = p1[:, :128]
    p12_ref[1] = p1[:, 128:]
    p12_ref[2] = p2[:, :128]
    p12_ref[3] = p2[:, 128:]


def _k6_update(s, aggs, degp, ones32, Wu, bu, W1, W2):
    full = lambda shape: pl.BlockSpec(shape, lambda i: tuple(0 for _ in shape))
    return pl.pallas_call(
        _k6_body,
        grid=(_NP // _NB,),
        in_specs=[pl.BlockSpec((_NB, _SD), lambda i: (i, 0)),
                  pl.BlockSpec((2, _NB, 128), lambda i: (0, i, 0)),
                  pl.BlockSpec((32, _NB), lambda i: (0, i)),
                  full((32, 1)),
                  full((2 * _SD, _SD)), full((1, _SD)),
                  full((_SD, _SD)), full((_SD, _SD))],
        out_specs=[pl.BlockSpec((_NB, _SD), lambda i: (i, 0)),
                   pl.BlockSpec((4, _NB, 128), lambda i: (0, i, 0))],
        out_shape=[jax.ShapeDtypeStruct((_NP, _SD), jnp.float32),
                   jax.ShapeDtypeStruct((4, _NP, 128), jnp.float32)],
    )(s, aggs, degp, ones32, Wu, bu, W1, W2)


def _k6l_body(s_blk, ag_blk, dg_blk, ones32, Wu, bu, sn_ref):
    deg = lax.dot_general(dg_blk[...], ones32[...], (((0,), (0,)), ((), ())),
                          preferred_element_type=jnp.float32)
    deginv = 1.0 / jnp.maximum(deg, 1.0)
    agg = jnp.concatenate([ag_blk[0], ag_blk[1]], axis=1) * deginv
    cat = jnp.concatenate([s_blk[...], agg], axis=1)
    upd = jax.nn.silu(jnp.dot(cat, Wu[...],
                              preferred_element_type=jnp.float32) + bu[...])
    sn_ref[...] = s_blk[...] + upd


def _k6_update_last(s, aggs, degp, ones32, Wu, bu):
    full = lambda shape: pl.BlockSpec(shape, lambda i: tuple(0 for _ in shape))
    return pl.pallas_call(
        _k6l_body,
        grid=(_NP // _NB,),
        in_specs=[pl.BlockSpec((_NB, _SD), lambda i: (i, 0)),
                  pl.BlockSpec((2, _NB, 128), lambda i: (0, i, 0)),
                  pl.BlockSpec((32, _NB), lambda i: (0, i)),
                  full((32, 1)),
                  full((2 * _SD, _SD)), full((1, _SD))],
        out_specs=pl.BlockSpec((_NB, _SD), lambda i: (i, 0)),
        out_shape=jax.ShapeDtypeStruct((_NP, _SD), jnp.float32),
    )(s, aggs, degp, ones32, Wu, bu)


def _k7_body(s_blk, b_blk, cnt_t, Wh1, bh1, Wh2, bh2, out_ref, acc_ref):
    i = pl.program_id(0)
    oh = (b_blk[...] == lax.broadcasted_iota(jnp.int32, (_NB, _G), 1)
          ).astype(jnp.float32)
    part = lax.dot_general(oh, s_blk[...], (((0,), (0,)), ((), ())),
                           preferred_element_type=jnp.float32)

    @pl.when(i == 0)
    def _():
        acc_ref[...] = part

    @pl.when(i != 0)
    def _():
        acc_ref[...] = acc_ref[...] + part

    @pl.when(i == _NP // _NB - 1)
    def _():
        gs = acc_ref[...] / jnp.maximum(cnt_t[...], 1.0)
        h = jax.nn.silu(jnp.dot(gs, Wh1[...],
                                preferred_element_type=jnp.float32) + bh1[...])
        out_ref[...] = jnp.dot(h, Wh2[...],
                               preferred_element_type=jnp.float32) + bh2[...]


def _k7_read(s, batch2, cnt_t, Wh1, bh1, Wh2, bh2):
    full = lambda shape: pl.BlockSpec(shape, lambda i: tuple(0 for _ in shape))
    return pl.pallas_call(
        _k7_body,
        grid=(_NP // _NB,),
        in_specs=[pl.BlockSpec((_NB, _SD), lambda i: (i, 0)),
                  pl.BlockSpec((_NB, 1), lambda i: (i, 0)),
                  full((_G, 1)),
                  full((_SD, _SD)), full((1, _SD)),
                  full((_SD, 1)), full((1, 1))],
        out_specs=pl.BlockSpec((_G, 1), lambda i: (0, 0)),
        out_shape=jax.ShapeDtypeStruct((_G, 1), jnp.float32),
        scratch_shapes=[pltpu.VMEM((_G, _SD), jnp.float32)],
    )(s, batch2, cnt_t, Wh1, bh1, Wh2, bh2)


def kernel(x, t, pos, edge_index_local, edge_index_global, edge_attr_global,
           batch, batch_edge_global, params):
    del edge_index_local
    p = params
    row = lambda v: v.reshape(1, -1)

    x_p = jnp.pad(x, ((0, _NP - _N), (0, 0)))
    batch2 = jnp.pad(batch.astype(jnp.int32), (0, _NP - _N),
                     constant_values=1000).reshape(_NP, 1)
    pos_t = jnp.pad(pos.T, ((0, 0), (0, _NP - _N)))
    src = edge_index_global[0].astype(jnp.int32)
    dst = edge_index_global[1].astype(jnp.int32)
    src_p = jnp.pad(src, (0, _EP - _E))
    dst_p = jnp.pad(dst, (0, _EP - _E))
    src_r = src_p.reshape(16, _NCH, _CH)
    dst_r = dst_p.reshape(16, _NCH, _CH)
    gidx = jnp.stack(
        [jnp.stack([src_r + c * _NP, dst_r + 2 * _NP + c * _NP], axis=2)
         for c in (0, 1)], axis=0).reshape(-1)
    be2 = jnp.pad(batch_edge_global.astype(jnp.int32), (0, _EP - _E),
                  constant_values=1000).reshape(_EP, 1)
    ea8 = jnp.pad(edge_attr_global, ((0, _EP - _E), (0, 3)))
    ones32 = jnp.ones((32, 1), jnp.float32)

    cnt, ps_t = _k1_stats(pos_t, batch2)

    lay0 = p['layers'][0]
    s, posct, p12 = _k2_node(
        x_p, batch2, pos_t, t, cnt, ps_t,
        p['Wta'], row(p['bta']), p['Wam'], row(p['bam']),
        p['Wat'], row(p['bat']),
        lay0['Wm'][:_SD], lay0['Wm'][_SD:2 * _SD])

    d2, av, degp = _k3_geom(posct[0], posct[1], posct[2], src, dst)
    degp = degp.reshape(32, _NP)
    d2c = jnp.pad(d2, (0, _EP - _E)).reshape(_EP, 1)
    ac = jnp.pad(av, (0, _EP - _E)).reshape(_EP, 1)

    cs = []
    for lay in p['layers']:
        Wm = lay['Wm']
        cl = _k4_cterm(
            ea8, d2c, ac, be2, t, p['Wtb'], row(p['btb']),
            jnp.pad(p['Wbm'], ((0, 3), (0, 0))), row(p['bbm']),
            p['Wbt'], row(p['bbt']),
            Wm[2 * _SD:2 * _SD + _ED], Wm[2 * _SD + _ED:2 * _SD + _ED + 1],
            Wm[2 * _SD + _ED + 1:2 * _SD + _ED + 2], row(lay['bm']))
        cs.append(cl.reshape(2 * _EP, 128))

    for li, lay in enumerate(p['layers']):
        aggs = _k5_edge(p12.reshape(4 * _NP, 128), cs[li], gidx, dst_p)
        if li + 1 < len(p['layers']):
            nxt = p['layers'][li + 1]
            s, p12 = _k6_update(
                s, aggs, degp, ones32, lay['Wu'], row(lay['bu']),
                nxt['Wm'][:_SD], nxt['Wm'][_SD:2 * _SD])
        else:
            s = _k6_update_last(s, aggs, degp, ones32, lay['Wu'],
                                row(lay['bu']))

    return _k7_read(s, batch2, cnt.T, p['Wh1'], row(p['bh1']),
                    p['Wh2'], row(p['bh2']))


# speedup vs baseline: 1.1426x; 1.1426x over previous
import functools

import jax
import jax.numpy as jnp
from jax import lax
from jax.experimental import pallas as pl
from jax.experimental.pallas import tpu as pltpu
from jax.experimental.pallas import tpu_sc as plsc

_N = 10000
_E = 320000
_G = 64
_NP = 10240
_SD = 256
_ED = 32

_NB = 2048
_EB = 2048
_EP = 329728
_EPW = _E // 32
_EPT = _EP // 16
_CH = 56
_NCH = _EPT // _CH
_ROWS_PT = _NP // 16


def _mesh():
    return plsc.VectorSubcoreMesh(core_axis_name="c", subcore_axis_name="s")


def _k1_body(pt_blk, b_blk, cnt_ref, ps_ref):
    i = pl.program_id(0)
    oh = (b_blk[...] == lax.broadcasted_iota(jnp.int32, (_NB, _G), 1)
          ).astype(jnp.float32)
    cnt_p = jnp.sum(oh, axis=0, keepdims=True)
    ps_p = lax.dot_general(pt_blk[...], oh, (((1,), (0,)), ((), ())),
                           preferred_element_type=jnp.float32)

    @pl.when(i == 0)
    def _():
        cnt_ref[...] = cnt_p
        ps_ref[...] = ps_p

    @pl.when(i != 0)
    def _():
        cnt_ref[...] = cnt_ref[...] + cnt_p
        ps_ref[...] = ps_ref[...] + ps_p


def _k1_stats(pos_t, batch2):
    return pl.pallas_call(
        _k1_body,
        grid=(_NP // _NB,),
        in_specs=[pl.BlockSpec((3, _NB), lambda i: (0, i)),
                  pl.BlockSpec((_NB, 1), lambda i: (i, 0))],
        out_specs=[pl.BlockSpec((1, _G), lambda i: (0, 0)),
                   pl.BlockSpec((3, _G), lambda i: (0, 0))],
        out_shape=[jax.ShapeDtypeStruct((1, _G), jnp.float32),
                   jax.ShapeDtypeStruct((3, _G), jnp.float32)],
    )(pos_t, batch2)


def _k2_body(x_blk, b_blk, pt_blk, t_ref, cnt_ref, ps_ref, Wta, bta, Wam,
             bam, Wat, bat, W1, W2, s0_ref, pct_ref, p1_ref, p2_ref):
    oh = (b_blk[...] == lax.broadcasted_iota(jnp.int32, (_NB, _G), 1)
          ).astype(jnp.float32)
    meanp = ps_ref[...] / jnp.maximum(cnt_ref[...], 1.0)
    pct_ref[...] = pt_blk[...] - lax.dot_general(
        meanp, oh, (((1,), (1,)), ((), ())),
        preferred_element_type=jnp.float32)
    ta = jnp.dot(t_ref[...], Wta[...],
                 preferred_element_type=jnp.float32) + bta[...]
    tn = jnp.dot(oh, ta, preferred_element_type=jnp.float32)
    s_pre = (jnp.dot(x_blk[...], Wam[...],
                     preferred_element_type=jnp.float32) + bam[...] + tn)
    s0 = jnp.dot(s_pre, Wat[...],
                 preferred_element_type=jnp.float32) + bat[...]
    s0_ref[...] = s0
    p1 = jnp.dot(s0, W1[...], preferred_element_type=jnp.float32)
    p2 = jnp.dot(s0, W2[...], preferred_element_type=jnp.float32)
    p1_ref[0] = p1[:, :128]
    p1_ref[1] = p1[:, 128:]
    p2_ref[0] = p2[:, :128]
    p2_ref[1] = p2[:, 128:]


def _k2_node(x_p, batch2, pos_t, t, cnt, ps_t, Wta, bta, Wam, bam, Wat, bat,
             W1, W2):
    full = lambda shape: pl.BlockSpec(shape, lambda i: tuple(0 for _ in shape))
    return pl.pallas_call(
        _k2_body,
        grid=(_NP // _NB,),
        in_specs=[pl.BlockSpec((_NB, 128), lambda i: (i, 0)),
                  pl.BlockSpec((_NB, 1), lambda i: (i, 0)),
                  pl.BlockSpec((3, _NB), lambda i: (0, i)),
                  full((_G, 1)), full((1, _G)), full((3, _G)),
                  full((1, _SD)), full((1, _SD)),
                  full((128, _SD)), full((1, _SD)),
                  full((_SD, _SD)), full((1, _SD)),
                  full((_SD, _SD)), full((_SD, _SD))],
        out_specs=[pl.BlockSpec((_NB, _SD), lambda i: (i, 0)),
                   pl.BlockSpec((3, _NB), lambda i: (0, i)),
                   pl.BlockSpec((2, _NB, 128), lambda i: (0, i, 0)),
                   pl.BlockSpec((2, _NB, 128), lambda i: (0, i, 0))],
        out_shape=[jax.ShapeDtypeStruct((_NP, _SD), jnp.float32),
                   jax.ShapeDtypeStruct((3, _NP), jnp.float32),
                   jax.ShapeDtypeStruct((2, _NP, 128), jnp.float32),
                   jax.ShapeDtypeStruct((2, _NP, 128), jnp.float32)],
    )(x_p, batch2, pos_t, t, cnt, ps_t, Wta, bta, Wam, bam, Wat, bat, W1, W2)


def _geom_body(posx, posy, posz, src_h, dst_h, d2_h, a_h, degp_h, px, py, pz,
               srcv, dstv, d2v, av, degv):
    c = lax.axis_index("c")
    s = lax.axis_index("s")
    w = s * 2 + c
    pltpu.sync_copy(posx, px)
    pltpu.sync_copy(posy, py)
    pltpu.sync_copy(posz, pz)
    base = w * _EPW
    pltpu.sync_copy(src_h.at[pl.ds(base, _EPW)], srcv)
    pltpu.sync_copy(dst_h.at[pl.ds(base, _EPW)], dstv)
    zeros16 = jnp.zeros((16,), jnp.float32)

    def zbody(i, carry):
        degv[pl.ds(i * 16, 16)] = zeros16
        return carry

    lax.fori_loop(0, _NP // 16, zbody, 0)
    ones16 = jnp.ones((16,), jnp.float32)

    def body(j, carry):
        si = srcv[pl.ds(j * 16, 16)]
        di = dstv[pl.ds(j * 16, 16)]
        xs = plsc.load_gather(px, [si])
        ys = plsc.load_gather(py, [si])
        zs = plsc.load_gather(pz, [si])
        xd = plsc.load_gather(px, [di])
        yd = plsc.load_gather(py, [di])
        zd = plsc.load_gather(pz, [di])
        dx = xd - xs
        dy = yd - ys
        dz = zd - zs
        d2v[pl.ds(j * 16, 16)] = dx * dx + dy * dy + dz * dz
        av[pl.ds(j * 16, 16)] = xs * xd + ys * yd + zs * zd
        plsc.addupdate_scatter(degv, [di], ones16)
        return carry

    lax.fori_loop(0, _EPW // 16, body, 0)
    pltpu.sync_copy(d2v, d2_h.at[pl.ds(base, _EPW)])
    pltpu.sync_copy(av, a_h.at[pl.ds(base, _EPW)])
    pltpu.sync_copy(degv, degp_h.at[pl.ds(w * _NP, _NP)])


def _k3_geom(posx, posy, posz, src, dst):
    fn = functools.partial(
        pl.kernel,
        out_type=(jax.ShapeDtypeStruct((_E,), jnp.float32),
                  jax.ShapeDtypeStruct((_E,), jnp.float32),
                  jax.ShapeDtypeStruct((32 * _NP,), jnp.float32)),
        mesh=_mesh(),
        compiler_params=pltpu.CompilerParams(needs_layout_passes=False),
        scratch_types=[pltpu.VMEM((_NP,), jnp.float32),
                       pltpu.VMEM((_NP,), jnp.float32),
                       pltpu.VMEM((_NP,), jnp.float32),
                       pltpu.VMEM((_EPW,), jnp.int32),
                       pltpu.VMEM((_EPW,), jnp.int32),
                       pltpu.VMEM((_EPW,), jnp.float32),
                       pltpu.VMEM((_EPW,), jnp.float32),
                       pltpu.VMEM((_NP,), jnp.float32)],
    )(_geom_body)
    return fn(posx, posy, posz, src, dst)


def _k4_body(ea_blk, d2_blk, a_blk, be_blk, t_ref, Wtb, btb, Wbm, bbm, Wbt,
             bbt, WmE, wd, wa, bm, c_ref):
    oh = (be_blk[...] == lax.broadcasted_iota(jnp.int32, (_EB, _G), 1)
          ).astype(jnp.float32)
    tb = jnp.dot(t_ref[...], Wtb[...],
                 preferred_element_type=jnp.float32) + btb[...]
    te = jnp.dot(oh, tb, preferred_element_type=jnp.float32)
    e1 = (jnp.dot(ea_blk[...], Wbm[...],
                  preferred_element_type=jnp.float32) + bbm[...] + te)
    e2 = jnp.dot(e1, Wbt[...], preferred_element_type=jnp.float32) + bbt[...]
    dd = jnp.sqrt(jnp.maximum(d2_blk[...], 1e-6))
    cc = (jnp.dot(e2, WmE[...], preferred_element_type=jnp.float32)
          + dd * wd[...] + a_blk[...] * wa[...] + bm[...])
    rid = (pl.program_id(1) * _EB
           + lax.broadcasted_iota(jnp.int32, (_EB, 1), 0))
    c_ref[0] = jnp.where(rid < _E, cc, -1e6)


def _k4_cterm(ea, d2c, ac, be2, t, Wtb, btb, Wbm, bbm, Wbt, bbt, WmE, wd, wa,
              bm):
    full = lambda shape: pl.BlockSpec(shape, lambda c, i: tuple(0 for _ in shape))
    return pl.pallas_call(
        _k4_body,
        grid=(2, _EP // _EB),
        in_specs=[pl.BlockSpec((_EB, 8), lambda c, i: (i, 0)),
                  pl.BlockSpec((_EB, 1), lambda c, i: (i, 0)),
                  pl.BlockSpec((_EB, 1), lambda c, i: (i, 0)),
                  pl.BlockSpec((_EB, 1), lambda c, i: (i, 0)),
                  full((_G, 1)), full((1, _ED)), full((1, _ED)),
                  full((8, _ED)), full((1, _ED)),
                  full((_ED, _ED)), full((1, _ED)),
                  pl.BlockSpec((_ED, 128), lambda c, i: (0, c)),
                  pl.BlockSpec((1, 128), lambda c, i: (0, c)),
                  pl.BlockSpec((1, 128), lambda c, i: (0, c)),
                  pl.BlockSpec((1, 128), lambda c, i: (0, c))],
        out_specs=pl.BlockSpec((1, _EB, 128), lambda c, i: (c, i, 0)),
        out_shape=jax.ShapeDtypeStruct((2, _EP, 128), jnp.float32),
    )(ea, d2c, ac, be2, t, Wtb, btb, Wbm, bbm, Wbt, bbt, WmE, wd, wa, bm)


def _edge_body(p1f, p2f, clf, src2, dst2, dstp, out, si0, si1, di0, di1,
               dp0, dp1, b1a, b1b, b2a, b2b, bca, bcb, agg, sx0, sx1, sg0,
               sg1):
    c = lax.axis_index("c")
    s = lax.axis_index("s")
    sis = (si0, si1)
    dis = (di0, di1)
    dps = (dp0, dp1)
    b1s = (b1a, b1b)
    b2s = (b2a, b2b)
    bcs = (bca, bcb)
    sxs = (sx0, sx1)
    sgs = (sg0, sg1)
    zrow = jnp.zeros((16,), jnp.float32)

    def zb_body(r, carry):
        for q in range(8):
            b1a[r, pl.ds(q * 16, 16)] = zrow
        return carry

    lax.fori_loop(0, _CH, zb_body, 0)
    rbase = s * _ROWS_PT
    for i in range(_ROWS_PT // _CH):
        pltpu.sync_copy(b1a, agg.at[pl.ds(rbase + i * _CH, _CH)])
    _rem = _ROWS_PT - (_ROWS_PT // _CH) * _CH
    pltpu.sync_copy(b1a.at[pl.ds(0, _rem)],
                    agg.at[pl.ds(rbase + (_ROWS_PT // _CH) * _CH, _rem)])
    plsc.subcore_barrier()
    ebase = s * _EPT
    coff = c * _EP

    def idx_start(k, sb):
        pltpu.make_async_copy(src2.at[pl.ds(coff + ebase + k * _CH, _CH)],
                              sis[sb], sxs[sb]).start()
        pltpu.make_async_copy(dst2.at[pl.ds(coff + ebase + k * _CH, _CH)],
                              dis[sb], sxs[sb]).start()
        pltpu.make_async_copy(dstp.at[pl.ds(ebase + k * _CH, _CH)],
                              dps[sb], sxs[sb]).start()

    def idx_wait(sb):
        pltpu.make_async_copy(dstp.at[pl.ds(0, _CH)], sis[sb],
                              sxs[sb]).wait()
        pltpu.make_async_copy(dstp.at[pl.ds(0, _CH)], dis[sb],
                              sxs[sb]).wait()
        pltpu.make_async_copy(dstp.at[pl.ds(0, _CH)], dps[sb],
                              sxs[sb]).wait()

    def gather_start(k, sb):
        pltpu.make_async_copy(p1f.at[sis[sb]], b1s[sb], sgs[sb]).start()
        pltpu.make_async_copy(p2f.at[dis[sb]], b2s[sb], sgs[sb]).start()
        pltpu.make_async_copy(clf.at[pl.ds(coff + ebase + k * _CH, _CH)],
                              bcs[sb], sgs[sb]).start()

    def gather_wait(sb):
        pltpu.make_async_copy(clf.at[pl.ds(0, _CH)], bcs[sb],
                              sgs[sb]).wait()
        pltpu.make_async_copy(clf.at[pl.ds(0, _CH)], b1s[sb],
                              sgs[sb]).wait()
        pltpu.make_async_copy(clf.at[pl.ds(0, _CH)], b2s[sb],
                              sgs[sb]).wait()

    idx_start(0, 0)
    idx_wait(0)
    gather_start(0, 0)
    idx_start(1, 1)

    def outer(k0, carry):
        for b in range(2):
            k = k0 * 2 + b
            nk = k + 1

            @pl.when(nk < _NCH)
            def _():
                idx_wait(1 - b)
                gather_start(nk, 1 - b)

            gather_wait(b)

            def crow(r, carry2):
                for q in range(8):
                    x = (b1s[b][r, pl.ds(q * 16, 16)]
                         + b2s[b][r, pl.ds(q * 16, 16)]
                         + bcs[b][r, pl.ds(q * 16, 16)])
                    x = jnp.maximum(x, -60.0)
                    b1s[b][r, pl.ds(q * 16, 16)] = x / (1.0 + jnp.exp(-x))
                return carry2

            lax.fori_loop(0, _CH, crow, 0)
            pltpu.sync_copy(b1s[b], agg.at[dps[b]], add=True)

            @pl.when(k + 2 < _NCH)
            def _():
                idx_start(k + 2, b)
        return carry

    lax.fori_loop(0, _NCH // 2, outer, 0)
    plsc.subcore_barrier()
    pltpu.sync_copy(agg.at[pl.ds(rbase, _ROWS_PT)],
                    out.at[c, pl.ds(rbase, _ROWS_PT)])


def _k5_edge(p1f, p2f, clf, src2, dst2, dstp):
    fn = functools.partial(
        pl.kernel,
        out_type=jax.ShapeDtypeStruct((2, _NP, 128), jnp.float32),
        mesh=_mesh(),
        compiler_params=pltpu.CompilerParams(needs_layout_passes=False),
        scratch_types=[pltpu.VMEM((_CH,), jnp.int32),
                       pltpu.VMEM((_CH,), jnp.int32),
                       pltpu.VMEM((_CH,), jnp.int32),
                       pltpu.VMEM((_CH,), jnp.int32),
                       pltpu.VMEM((_CH,), jnp.int32),
                       pltpu.VMEM((_CH,), jnp.int32),
                       pltpu.VMEM((_CH, 128), jnp.float32),
                       pltpu.VMEM((_CH, 128), jnp.float32),
                       pltpu.VMEM((_CH, 128), jnp.float32),
                       pltpu.VMEM((_CH, 128), jnp.float32),
                       pltpu.VMEM((_CH, 128), jnp.float32),
                       pltpu.VMEM((_CH, 128), jnp.float32),
                       pltpu.VMEM_SHARED((_NP, 128), jnp.float32),
                       pltpu.SemaphoreType.DMA,
                       pltpu.SemaphoreType.DMA,
                       pltpu.SemaphoreType.DMA,
                       pltpu.SemaphoreType.DMA],
    )(_edge_body)
    return fn(p1f, p2f, clf, src2, dst2, dstp)


def _k6_body(s_blk, ag_blk, dg_blk, ones32, Wu, bu, W1, W2, sn_ref, p1_ref,
             p2_ref):
    deg = lax.dot_general(dg_blk[...], ones32[...], (((0,), (0,)), ((), ())),
                          preferred_element_type=jnp.float32)
    deginv = 1.0 / jnp.maximum(deg, 1.0)
    agg = jnp.concatenate([ag_blk[0], ag_blk[1]], axis=1) * deginv
    cat = jnp.concatenate([s_blk[...], agg], axis=1)
    upd = jax.nn.silu(jnp.dot(cat, Wu[...],
                              preferred_element_type=jnp.float32) + bu[...])
    sn = s_blk[...] + upd
    sn_ref[...] = sn
    p1 = jnp.dot(sn, W1[...], preferred_element_type=jnp.float32)
    p2 = jnp.dot(sn, W2[...], preferred_element_type=jnp.float32)
    p1_ref[0] = p1[:, :128]
    p1_ref[1] = p1[:, 128:]
    p2_ref[0] = p2[:, :128]
    p2_ref[1] = p2[:, 128:]


def _k6_update(s, aggs, degp, ones32, Wu, bu, W1, W2):
    full = lambda shape: pl.BlockSpec(shape, lambda i: tuple(0 for _ in shape))
    return pl.pallas_call(
        _k6_body,
        grid=(_NP // _NB,),
        in_specs=[pl.BlockSpec((_NB, _SD), lambda i: (i, 0)),
                  pl.BlockSpec((2, _NB, 128), lambda i: (0, i, 0)),
                  pl.BlockSpec((32, _NB), lambda i: (0, i)),
                  full((32, 1)),
                  full((2 * _SD, _SD)), full((1, _SD)),
                  full((_SD, _SD)), full((_SD, _SD))],
        out_specs=[pl.BlockSpec((_NB, _SD), lambda i: (i, 0)),
                   pl.BlockSpec((2, _NB, 128), lambda i: (0, i, 0)),
                   pl.BlockSpec((2, _NB, 128), lambda i: (0, i, 0))],
        out_shape=[jax.ShapeDtypeStruct((_NP, _SD), jnp.float32),
                   jax.ShapeDtypeStruct((2, _NP, 128), jnp.float32),
                   jax.ShapeDtypeStruct((2, _NP, 128), jnp.float32)],
    )(s, aggs, degp, ones32, Wu, bu, W1, W2)


def _k6l_body(s_blk, ag_blk, dg_blk, ones32, Wu, bu, sn_ref):
    deg = lax.dot_general(dg_blk[...], ones32[...], (((0,), (0,)), ((), ())),
                          preferred_element_type=jnp.float32)
    deginv = 1.0 / jnp.maximum(deg, 1.0)
    agg = jnp.concatenate([ag_blk[0], ag_blk[1]], axis=1) * deginv
    cat = jnp.concatenate([s_blk[...], agg], axis=1)
    upd = jax.nn.silu(jnp.dot(cat, Wu[...],
                              preferred_element_type=jnp.float32) + bu[...])
    sn_ref[...] = s_blk[...] + upd


def _k6_update_last(s, aggs, degp, ones32, Wu, bu):
    full = lambda shape: pl.BlockSpec(shape, lambda i: tuple(0 for _ in shape))
    return pl.pallas_call(
        _k6l_body,
        grid=(_NP // _NB,),
        in_specs=[pl.BlockSpec((_NB, _SD), lambda i: (i, 0)),
                  pl.BlockSpec((2, _NB, 128), lambda i: (0, i, 0)),
                  pl.BlockSpec((32, _NB), lambda i: (0, i)),
                  full((32, 1)),
                  full((2 * _SD, _SD)), full((1, _SD))],
        out_specs=pl.BlockSpec((_NB, _SD), lambda i: (i, 0)),
        out_shape=jax.ShapeDtypeStruct((_NP, _SD), jnp.float32),
    )(s, aggs, degp, ones32, Wu, bu)


def _k7_body(s_blk, b_blk, cnt_t, Wh1, bh1, Wh2, bh2, out_ref, acc_ref):
    i = pl.program_id(0)
    oh = (b_blk[...] == lax.broadcasted_iota(jnp.int32, (_NB, _G), 1)
          ).astype(jnp.float32)
    part = lax.dot_general(oh, s_blk[...], (((0,), (0,)), ((), ())),
                           preferred_element_type=jnp.float32)

    @pl.when(i == 0)
    def _():
        acc_ref[...] = part

    @pl.when(i != 0)
    def _():
        acc_ref[...] = acc_ref[...] + part

    @pl.when(i == _NP // _NB - 1)
    def _():
        gs = acc_ref[...] / jnp.maximum(cnt_t[...], 1.0)
        h = jax.nn.silu(jnp.dot(gs, Wh1[...],
                                preferred_element_type=jnp.float32) + bh1[...])
        out_ref[...] = jnp.dot(h, Wh2[...],
                               preferred_element_type=jnp.float32) + bh2[...]


def _k7_read(s, batch2, cnt_t, Wh1, bh1, Wh2, bh2):
    full = lambda shape: pl.BlockSpec(shape, lambda i: tuple(0 for _ in shape))
    return pl.pallas_call(
        _k7_body,
        grid=(_NP // _NB,),
        in_specs=[pl.BlockSpec((_NB, _SD), lambda i: (i, 0)),
                  pl.BlockSpec((_NB, 1), lambda i: (i, 0)),
                  full((_G, 1)),
                  full((_SD, _SD)), full((1, _SD)),
                  full((_SD, 1)), full((1, 1))],
        out_specs=pl.BlockSpec((_G, 1), lambda i: (0, 0)),
        out_shape=jax.ShapeDtypeStruct((_G, 1), jnp.float32),
        scratch_shapes=[pltpu.VMEM((_G, _SD), jnp.float32)],
    )(s, batch2, cnt_t, Wh1, bh1, Wh2, bh2)


def kernel(x, t, pos, edge_index_local, edge_index_global, edge_attr_global,
           batch, batch_edge_global, params):
    del edge_index_local
    p = params
    row = lambda v: v.reshape(1, -1)

    x_p = jnp.pad(x, ((0, _NP - _N), (0, 0)))
    batch2 = jnp.pad(batch.astype(jnp.int32), (0, _NP - _N),
                     constant_values=1000).reshape(_NP, 1)
    pos_t = jnp.pad(pos.T, ((0, 0), (0, _NP - _N)))
    src = edge_index_global[0].astype(jnp.int32)
    dst = edge_index_global[1].astype(jnp.int32)
    src_p = jnp.pad(src, (0, _EP - _E))
    dst_p = jnp.pad(dst, (0, _EP - _E))
    src2 = jnp.concatenate([src_p, src_p + _NP])
    dst2 = jnp.concatenate([dst_p, dst_p + _NP])
    be2 = jnp.pad(batch_edge_global.astype(jnp.int32), (0, _EP - _E),
                  constant_values=1000).reshape(_EP, 1)
    ea8 = jnp.pad(edge_attr_global, ((0, _EP - _E), (0, 3)))
    ones32 = jnp.ones((32, 1), jnp.float32)

    cnt, ps_t = _k1_stats(pos_t, batch2)

    lay0 = p['layers'][0]
    s, posct, p1, p2 = _k2_node(
        x_p, batch2, pos_t, t, cnt, ps_t,
        p['Wta'], row(p['bta']), p['Wam'], row(p['bam']),
        p['Wat'], row(p['bat']),
        lay0['Wm'][:_SD], lay0['Wm'][_SD:2 * _SD])

    d2, av, degp = _k3_geom(posct[0], posct[1], posct[2], src, dst)
    degp = degp.reshape(32, _NP)
    d2c = jnp.pad(d2, (0, _EP - _E)).reshape(_EP, 1)
    ac = jnp.pad(av, (0, _EP - _E)).reshape(_EP, 1)

    cs = []
    for lay in p['layers']:
        Wm = lay['Wm']
        cl = _k4_cterm(
            ea8, d2c, ac, be2, t, p['Wtb'], row(p['btb']),
            jnp.pad(p['Wbm'], ((0, 3), (0, 0))), row(p['bbm']),
            p['Wbt'], row(p['bbt']),
            Wm[2 * _SD:2 * _SD + _ED], Wm[2 * _SD + _ED:2 * _SD + _ED + 1],
            Wm[2 * _SD + _ED + 1:2 * _SD + _ED + 2], row(lay['bm']))
        cs.append(cl.reshape(2 * _EP, 128))

    for li, lay in enumerate(p['layers']):
        aggs = _k5_edge(p1.reshape(2 * _NP, 128), p2.reshape(2 * _NP, 128),
                        cs[li], src2, dst2, dst_p)
        if li + 1 < len(p['layers']):
            nxt = p['layers'][li + 1]
            s, p1, p2 = _k6_update(
                s, aggs, degp, ones32, lay['Wu'], row(lay['bu']),
                nxt['Wm'][:_SD], nxt['Wm'][_SD:2 * _SD])
        else:
            s = _k6_update_last(s, aggs, degp, ones32, lay['Wu'],
                                row(lay['bu']))

    return _k7_read(s, batch2, cnt.T, p['Wh1'], row(p['bh1']),
                    p['Wh2'], row(p['bh2']))
